# baseline (speedup 1.0000x reference)
import jax
import jax.numpy as jnp
from jax.experimental import pallas as pl
from jax.experimental.pallas import tpu as pltpu

_BM = 256


def _gcn_body(x_ref, w_ref, adj_ref, out_ref, x_vmem, wz_vmem):
    @pl.when(pl.program_id(0) == 0)
    def _():
        x_vmem[:, :128] = x_ref[0]
        x_vmem[:, 128:] = x_ref[1]
        w = w_ref[...]
        z = jnp.zeros_like(w)
        wz_vmem[...] = jnp.concatenate(
            [jnp.concatenate([w, z], axis=1), jnp.concatenate([z, w], axis=1)],
            axis=0,
        )

    t = jnp.dot(adj_ref[...], x_vmem[...], preferred_element_type=jnp.float32)
    acc = jnp.dot(t, wz_vmem[...], preferred_element_type=jnp.float32)
    out_ref[...] = jnp.maximum(acc, 0.0)


def kernel(input, adj, W):
    bs, n_agents, in_f = input.shape
    out_f = W.shape[1]

    grid = (n_agents // _BM,)
    out = pl.pallas_call(
        _gcn_body,
        grid=grid,
        in_specs=[
            pl.BlockSpec((bs, n_agents, in_f), lambda i: (0, 0, 0)),
            pl.BlockSpec((in_f, out_f), lambda i: (0, 0)),
            pl.BlockSpec((_BM, n_agents), lambda i: (i, 0)),
        ],
        out_specs=pl.BlockSpec((_BM, bs * out_f), lambda i: (i, 0)),
        out_shape=jax.ShapeDtypeStruct((n_agents, bs * out_f), jnp.float32),
        scratch_shapes=[
            pltpu.VMEM((n_agents, bs * in_f), jnp.float32),
            pltpu.VMEM((bs * in_f, bs * out_f), jnp.float32),
        ],
        compiler_params=pltpu.CompilerParams(
            dimension_semantics=("arbitrary",),
            vmem_limit_bytes=120 * 1024 * 1024,
        ),
    )(input, W, adj)

    out = out.reshape(n_agents, bs, out_f).transpose(1, 0, 2)
    return out.reshape(bs * n_agents, out_f)


# speedup vs baseline: 1.0297x; 1.0297x over previous
import jax
import jax.numpy as jnp
from jax.experimental import pallas as pl
from jax.experimental.pallas import tpu as pltpu

_BM = 256


def _gcn_body(x_ref, w_ref, adj_ref, out_ref, s_vmem):
    @pl.when(pl.program_id(0) == 0)
    def _():
        w = w_ref[...]
        s0 = jnp.dot(x_ref[0], w, preferred_element_type=jnp.float32)
        s1 = jnp.dot(x_ref[1], w, preferred_element_type=jnp.float32)
        s_vmem[...] = jnp.concatenate([s0, s1], axis=1)

    acc = jnp.dot(adj_ref[...], s_vmem[...], preferred_element_type=jnp.float32)
    out_ref[...] = jnp.maximum(acc, 0.0)


def kernel(input, adj, W):
    bs, n_agents, in_f = input.shape
    out_f = W.shape[1]

    grid = (n_agents // _BM,)
    out = pl.pallas_call(
        _gcn_body,
        grid=grid,
        in_specs=[
            pl.BlockSpec((bs, n_agents, in_f), lambda i: (0, 0, 0)),
            pl.BlockSpec((in_f, out_f), lambda i: (0, 0)),
            pl.BlockSpec((_BM, n_agents), lambda i: (i, 0)),
        ],
        out_specs=pl.BlockSpec((_BM, bs * out_f), lambda i: (i, 0)),
        out_shape=jax.ShapeDtypeStruct((n_agents, bs * out_f), jnp.float32),
        scratch_shapes=[pltpu.VMEM((n_agents, bs * out_f), jnp.float32)],
        compiler_params=pltpu.CompilerParams(
            dimension_semantics=("arbitrary",),
            vmem_limit_bytes=120 * 1024 * 1024,
        ),
    )(input, W, adj)

    out = out.reshape(n_agents, bs, out_f).transpose(1, 0, 2)
    return out.reshape(bs * n_agents, out_f)
